# baseline (speedup 1.0000x reference)
import functools

import jax
import jax.numpy as jnp
from jax import lax
from jax.experimental import pallas as pl
from jax.experimental.pallas import tpu as pltpu
from jax.experimental.pallas import tpu_sc as plsc

_NUM_CORES = 2
_NUM_SUBCORES = 16
_NUM_WORKERS = _NUM_CORES * _NUM_SUBCORES

_CHUNK = 32
_NBUF = 3


def _gather_rows(idx, table):
    b = idx.shape[0]
    d = table.shape[1]
    b_per_w = b // _NUM_WORKERS
    n_chunks = b_per_w // _CHUNK
    assert n_chunks >= 6 and (n_chunks - 5) % _NBUF == 0

    mesh = plsc.VectorSubcoreMesh(core_axis_name="c", subcore_axis_name="s")
    idx3 = idx.reshape(_NUM_WORKERS, n_chunks, _CHUNK)

    @functools.partial(
        pl.kernel,
        out_type=jax.ShapeDtypeStruct((b, d), jnp.float32),
        mesh=mesh,
        scratch_types=[
            pltpu.VMEM((n_chunks, _CHUNK), jnp.int32),
            pltpu.VMEM((_NBUF, _CHUNK, d), jnp.float32),
            pltpu.SemaphoreType.DMA,
            pltpu.SemaphoreType.DMA,
        ],
    )
    def sc_kernel(idx_hbm, table_hbm, out_hbm, idx_v, rows_v, gsem, ssem):
        wid = lax.axis_index("s") * _NUM_CORES + lax.axis_index("c")
        base = wid * b_per_w

        def gather(c, slot):
            pltpu.async_copy(table_hbm.at[idx_v.at[c]], rows_v.at[slot], gsem)

        def wait_gather(slot):
            pltpu.make_async_copy(
                table_hbm.at[idx_v.at[0]], rows_v.at[slot], gsem
            ).wait()

        def scatter(c, slot):
            pltpu.async_copy(
                rows_v.at[slot], out_hbm.at[pl.ds(base + c * _CHUNK, _CHUNK)], ssem
            )

        def drain_one_scatter():
            pltpu.make_async_copy(
                rows_v.at[0], out_hbm.at[pl.ds(base, _CHUNK)], ssem
            ).wait()

        pltpu.sync_copy(idx_hbm.at[wid], idx_v)

        gather(0, 0)
        gather(1, 1)
        wait_gather(0)
        scatter(0, 0)
        gather(2, 2)
        wait_gather(1)
        scatter(1, 1)
        drain_one_scatter()
        gather(3, 0)

        for c in range(2, n_chunks - 3):
            wait_gather(c % _NBUF)
            scatter(c, c % _NBUF)
            drain_one_scatter()
            gather(c + 2, (c + 2) % _NBUF)

        c = n_chunks - 3
        wait_gather(c % _NBUF)
        scatter(c, c % _NBUF)
        drain_one_scatter()
        gather(c + 2, (c + 2) % _NBUF)
        wait_gather((c + 1) % _NBUF)
        scatter(c + 1, (c + 1) % _NBUF)
        wait_gather((c + 2) % _NBUF)
        scatter(c + 2, (c + 2) % _NBUF)
        for _unused in range(3):
            drain_one_scatter()

    return sc_kernel(idx3, table)


def kernel(timestep, pe_matrix):
    flat_idx = timestep.reshape(-1)
    out = _gather_rows(flat_idx, pe_matrix)
    return out.reshape(timestep.shape + (pe_matrix.shape[1],))


# speedup vs baseline: 1.0286x; 1.0286x over previous
import functools

import jax
import jax.numpy as jnp
from jax import lax
from jax.experimental import pallas as pl
from jax.experimental.pallas import tpu as pltpu
from jax.experimental.pallas import tpu_sc as plsc

_NUM_CORES = 2
_NUM_SUBCORES = 16
_NUM_WORKERS = _NUM_CORES * _NUM_SUBCORES

_CHUNK = 32
_NBUF = 3


def _gather_rows(idx, table):
    b = idx.shape[0]
    d = table.shape[1]
    b_per_w = b // _NUM_WORKERS
    n_chunks = b_per_w // _CHUNK
    assert n_chunks >= 6 and (n_chunks - 5) % _NBUF == 0

    mesh = plsc.VectorSubcoreMesh(core_axis_name="c", subcore_axis_name="s")
    idx3 = idx.reshape(_NUM_WORKERS, n_chunks, _CHUNK)

    @functools.partial(
        pl.kernel,
        out_type=jax.ShapeDtypeStruct((b, d), jnp.float32),
        mesh=mesh,
        scratch_types=[
            pltpu.VMEM((n_chunks, _CHUNK), jnp.int32),
            pltpu.VMEM((_NBUF, _CHUNK, d), jnp.float32),
            pltpu.SemaphoreType.DMA,
            pltpu.SemaphoreType.DMA,
        ],
    )
    def sc_kernel(idx_hbm, table_hbm, out_hbm, idx_v, rows_v, gsem, ssem):
        wid = lax.axis_index("s") * _NUM_CORES + lax.axis_index("c")
        base = wid * b_per_w

        def gather(c, slot):
            pltpu.async_copy(table_hbm.at[idx_v.at[c]], rows_v.at[slot], gsem)

        def wait_gather(slot):
            pltpu.make_async_copy(
                table_hbm.at[idx_v.at[0]], rows_v.at[slot], gsem
            ).wait()

        def scatter(c, slot):
            pltpu.async_copy(
                rows_v.at[slot], out_hbm.at[pl.ds(base + c * _CHUNK, _CHUNK)], ssem
            )

        def drain_one_scatter():
            pltpu.make_async_copy(
                rows_v.at[0], out_hbm.at[pl.ds(base, _CHUNK)], ssem
            ).wait()

        pltpu.sync_copy(idx_hbm.at[wid], idx_v)

        gather(0, 0)
        gather(1, 1)
        wait_gather(0)
        scatter(0, 0)
        gather(2, 2)
        wait_gather(1)
        scatter(1, 1)
        drain_one_scatter()
        gather(3, 0)

        @pl.loop(2, n_chunks - 3, step=_NBUF)
        def _(c0):
            for db in range(_NBUF):
                c = c0 + db
                slot_c = (2 + db) % _NBUF
                slot_n = (4 + db) % _NBUF
                wait_gather(slot_c)
                scatter(c, slot_c)
                drain_one_scatter()
                gather(c + 2, slot_n)

        c = n_chunks - 3
        wait_gather(c % _NBUF)
        scatter(c, c % _NBUF)
        drain_one_scatter()
        gather(c + 2, (c + 2) % _NBUF)
        wait_gather((c + 1) % _NBUF)
        scatter(c + 1, (c + 1) % _NBUF)
        wait_gather((c + 2) % _NBUF)
        scatter(c + 2, (c + 2) % _NBUF)
        for _unused in range(3):
            drain_one_scatter()

    return sc_kernel(idx3, table)


def kernel(timestep, pe_matrix):
    flat_idx = timestep.reshape(-1)
    out = _gather_rows(flat_idx, pe_matrix)
    return out.reshape(timestep.shape + (pe_matrix.shape[1],))
